# baseline (speedup 1.0000x reference)
import functools
import math

import numpy as np
import jax
import jax.numpy as jnp
from jax import lax
from jax.experimental import pallas as pl
from jax.experimental.pallas import tpu as pltpu
from jax.experimental.pallas import tpu_sc as plsc

MAX_LEN = 5000
D_MODEL = 1024
SCALE = math.sqrt(1024.0)
BATCH = 4
SEQ_LEN = 4096

NUM_WORKERS = 32
POS_PER_TILE = SEQ_LEN // NUM_WORKERS
CHUNK = 32
NCHUNK = POS_PER_TILE // CHUNK
LANES = 16
VPR = D_MODEL // LANES


def _make_pe_np():
    position = np.arange(SEQ_LEN, dtype=np.float32)[:, None]
    div_term = np.exp(
        np.arange(0, D_MODEL, 2, dtype=np.float32) * -(math.log(10000.0) / D_MODEL)
    )
    pe = np.zeros((SEQ_LEN, D_MODEL), dtype=np.float32)
    val = position * div_term[None, :]
    pe[:, 0::2] = np.sin(val)
    pe[:, 1::2] = np.cos(val)
    return pe


_PE = _make_pe_np()

_mesh = plsc.VectorSubcoreMesh(core_axis_name="c", subcore_axis_name="s")


@functools.partial(
    pl.kernel,
    out_type=jax.ShapeDtypeStruct((BATCH, SEQ_LEN, D_MODEL), jnp.float32),
    mesh=_mesh,
    scratch_types=[
        pltpu.VMEM((CHUNK,), jnp.int32),
        pltpu.VMEM((CHUNK, D_MODEL), jnp.float32),
        pltpu.VMEM((CHUNK, D_MODEL), jnp.float32),
        pltpu.SemaphoreType.DMA,
    ],
)
def _emb_pe(x_hbm, table_hbm, pe_hbm, out_hbm, idx_v, emb_v, pe_v, sem):
    wid = lax.axis_index("s") * 2 + lax.axis_index("c")
    base = wid * POS_PER_TILE
    for k in range(NCHUNK):
        off = base + k * CHUNK
        pltpu.sync_copy(pe_hbm.at[pl.ds(off, CHUNK)], pe_v)
        for b in range(BATCH):
            pltpu.sync_copy(x_hbm.at[b, pl.ds(off, CHUNK)], idx_v)
            pltpu.async_copy(table_hbm.at[idx_v], emb_v, sem).wait()

            def ew(i, _):
                r = i // VPR
                j = i % VPR
                sl = pl.ds(j * LANES, LANES)
                emb_v[r, sl] = emb_v[r, sl] * SCALE + pe_v[r, sl]
                return 0

            lax.fori_loop(0, CHUNK * VPR, ew, 0)
            pltpu.sync_copy(emb_v, out_hbm.at[b, pl.ds(off, CHUNK)])


def kernel(x, table):
    pe = jnp.asarray(_PE)
    return _emb_pe(x, table, pe)


# speedup vs baseline: 1.7744x; 1.7744x over previous
import functools
import math

import numpy as np
import jax
import jax.numpy as jnp
from jax import lax
from jax.experimental import pallas as pl
from jax.experimental.pallas import tpu as pltpu
from jax.experimental.pallas import tpu_sc as plsc

MAX_LEN = 5000
D_MODEL = 1024
SCALE = math.sqrt(1024.0)
BATCH = 4
SEQ_LEN = 4096

NUM_WORKERS = 32
POS_PER_TILE = SEQ_LEN // NUM_WORKERS
CHUNK = 32
NCHUNK = POS_PER_TILE // CHUNK
LANES = 16
VPR = D_MODEL // LANES


def _make_pe_np():
    position = np.arange(SEQ_LEN, dtype=np.float32)[:, None]
    div_term = np.exp(
        np.arange(0, D_MODEL, 2, dtype=np.float32) * -(math.log(10000.0) / D_MODEL)
    )
    pe = np.zeros((SEQ_LEN, D_MODEL), dtype=np.float32)
    val = position * div_term[None, :]
    pe[:, 0::2] = np.sin(val)
    pe[:, 1::2] = np.cos(val)
    return pe


_PE = _make_pe_np()

_mesh = plsc.VectorSubcoreMesh(core_axis_name="c", subcore_axis_name="s")


@functools.partial(
    pl.kernel,
    out_type=jax.ShapeDtypeStruct((BATCH, SEQ_LEN, D_MODEL), jnp.float32),
    mesh=_mesh,
    scratch_types=[
        pltpu.VMEM((CHUNK,), jnp.int32),
        pltpu.VMEM((CHUNK, D_MODEL), jnp.float32),
        pltpu.VMEM((CHUNK, D_MODEL), jnp.float32),
        pltpu.SemaphoreType.DMA,
    ],
)
def _emb_pe(x_hbm, table_hbm, pe_hbm, out_hbm, idx_v, emb_v, pe_v, sem):
    wid = lax.axis_index("s") * 2 + lax.axis_index("c")
    base = wid * POS_PER_TILE
    for k in range(NCHUNK):
        off = base + k * CHUNK
        pltpu.sync_copy(pe_hbm.at[pl.ds(off, CHUNK)], pe_v)
        for b in range(BATCH):
            pltpu.sync_copy(x_hbm.at[b, pl.ds(off, CHUNK)], idx_v)
            pltpu.async_copy(table_hbm.at[idx_v], emb_v, sem).wait()

            def ew(r, _):
                for j in range(VPR):
                    sl = pl.ds(j * LANES, LANES)
                    emb_v[r, sl] = emb_v[r, sl] * SCALE + pe_v[r, sl]
                return 0

            lax.fori_loop(0, CHUNK, ew, 0)
            pltpu.sync_copy(emb_v, out_hbm.at[b, pl.ds(off, CHUNK)])


def kernel(x, table):
    pe = jnp.asarray(_PE)
    return _emb_pe(x, table, pe)
